# baseline (speedup 1.0000x reference)
import jax
import jax.numpy as jnp
from jax import lax
from jax.experimental import pallas as pl
from jax.experimental.pallas import tpu as pltpu
from jax.experimental.pallas import tpu_sc as plsc

NUM_COMPS = 100000
NUM_ATTRS = 200
NUM_OBJS = 500
D = 128
L = 16
NC = 2
NS = 16
NW = NC * NS
CHUNK = 80
NCHUNKS = NUM_COMPS // CHUNK
NSLOTS = (NCHUNKS + NW - 1) // NW
RPB = 4


def _rsqrt16(x):
    xi = plsc.bitcast(x, jnp.int32)
    yi = jnp.int32(0x5F3759DF) - lax.shift_right_logical(xi, 1)
    y = plsc.bitcast(yi, jnp.float32)
    for _ in range(2):
        y = y * (1.5 - 0.5 * x * y * y)
    return y


def _body(text_hbm, w_hbm, comp_hbm, attr_hbm, obj_hbm, ai_hbm, oi_hbm,
          out_hbm, attr_v, obj_v,
          tx0, cp0, ou0, wv0, av0, ov0,
          tx1, cp1, ou1, wv1, av1, ov1,
          isem0, isem1, osem0, osem1):
    wid = lax.axis_index("s") * NC + lax.axis_index("c")

    pltpu.sync_copy(attr_hbm, attr_v)
    pltpu.sync_copy(obj_hbm, obj_v)

    bufs = ((tx0, cp0, ou0, wv0, av0, ov0, isem0, osem0),
            (tx1, cp1, ou1, wv1, av1, ov1, isem1, osem1))

    def start_in(s, b):
        tx, cp, _, wv, av, ov, isem, _ = bufs[b]
        cid = wid + NW * s
        base = cid * (CHUNK * D)
        sbase = cid * CHUNK
        pltpu.async_copy(text_hbm.at[pl.ds(base, CHUNK * D)], tx, isem)
        pltpu.async_copy(comp_hbm.at[pl.ds(base, CHUNK * D)], cp, isem)
        pltpu.async_copy(w_hbm.at[pl.ds(sbase, CHUNK)], wv, isem)
        pltpu.async_copy(ai_hbm.at[pl.ds(sbase, CHUNK)], av, isem)
        pltpu.async_copy(oi_hbm.at[pl.ds(sbase, CHUNK)], ov, isem)

    def wait_in(b):
        tx, cp, _, wv, av, ov, isem, _ = bufs[b]
        pltpu.make_async_copy(text_hbm.at[pl.ds(0, CHUNK * D)], tx, isem).wait()
        pltpu.make_async_copy(comp_hbm.at[pl.ds(0, CHUNK * D)], cp, isem).wait()
        pltpu.make_async_copy(w_hbm.at[pl.ds(0, CHUNK)], wv, isem).wait()
        pltpu.make_async_copy(ai_hbm.at[pl.ds(0, CHUNK)], av, isem).wait()
        pltpu.make_async_copy(oi_hbm.at[pl.ds(0, CHUNK)], ov, isem).wait()

    def start_out(s, b):
        ou, osem = bufs[b][2], bufs[b][7]
        base = (wid + NW * s) * (CHUNK * D)
        pltpu.async_copy(ou, out_hbm.at[pl.ds(base, CHUNK * D)], osem)

    def wait_out(b):
        ou, osem = bufs[b][2], bufs[b][7]
        pltpu.make_async_copy(ou, out_hbm.at[pl.ds(0, CHUNK * D)], osem).wait()

    def compute(b):
        tx, cp, ou, wv, av, ov, _, _ = bufs[b]

        def do_group(g, carry):
            w16 = wv[pl.ds(g * L, L)]
            ai16 = av[pl.ds(g * L, L)] * (D // 2)
            oi16 = ov[pl.ds(g * L, L)] * (D // 2)
            ws = [w16[j] for j in range(L)]
            obases = [oi16[j] for j in range(L)]
            abase0 = ai16[0]

            def unpack32(ref, off):
                return plsc.unpack(
                    plsc.bitcast(ref[pl.ds(off, L)], jnp.bfloat16),
                    format=plsc.PackFormat.INTERLEAVED,
                    preferred_element_type=jnp.float32)

            def block(j, shared_ar, abases):
                rows = tuple(range(j, j + RPB))
                hws = [0.5 * ws[r] for r in rows]
                rbs = [(g * L + r) * D for r in rows]
                us = [[] for _ in rows]
                accs = [jnp.zeros((L,), jnp.float32) for _ in rows]
                for m in range(D // 32):
                    obs = [unpack32(obj_v, obases[r] + m * L) for r in rows]
                    if shared_ar is None:
                        ars = [unpack32(attr_v, abases[r] + m * L)
                               for r in rows]
                    else:
                        ars = [shared_ar[m]] * RPB
                    for h in range(2):
                        for i, r in enumerate(rows):
                            off = rbs[i] + m * 32 + h * L
                            ta = tx[pl.ds(off, L)]
                            co = cp[pl.ds(off, L)]
                            u = ta + ws[r] * co + hws[i] * (ars[i][h] + obs[i][h])
                            accs[i] = accs[i] + u * u
                            us[i].append(u)
                ssqs = [jnp.maximum(jnp.sum(a), 1e-24) for a in accs]
                rvs = [_rsqrt16(jnp.full((L,), s, jnp.float32)) for s in ssqs]
                for k in range(D // L):
                    for i in range(len(rows)):
                        ou[pl.ds(rbs[i] + k * L, L)] = us[i][k] * rvs[i]

            same = jnp.all(ai16 == abase0)

            @pl.when(same)
            def _():
                sar = [unpack32(attr_v, abase0 + m * L)
                       for m in range(D // 32)]
                for j in range(0, L, RPB):
                    block(j, sar, None)

            @pl.when(jnp.logical_not(same))
            def _():
                abases = [ai16[j] for j in range(L)]
                for j in range(0, L, RPB):
                    block(j, None, abases)

            return carry

        lax.fori_loop(0, CHUNK // L, do_group, 0)

    def valid(s):
        return wid + NW * s < NCHUNKS

    start_in(0, 0)

    def pair(p, carry):
        s0 = 2 * p

        @pl.when(valid(s0 + 1))
        def _():
            start_in(s0 + 1, 1)

        @pl.when(valid(s0))
        def _():
            wait_in(0)

            @pl.when(p > 0)
            def _():
                wait_out(0)

            compute(0)
            start_out(s0, 0)

        @pl.when(valid(s0 + 2))
        def _():
            start_in(s0 + 2, 0)

        @pl.when(valid(s0 + 1))
        def _():
            wait_in(1)

            @pl.when(p > 0)
            def _():
                wait_out(1)

            compute(1)
            start_out(s0 + 1, 1)

        return carry

    lax.fori_loop(0, NSLOTS // 2, pair, 0)
    wait_out(0)
    wait_out(1)


def _pack_table(t):
    r = t.shape[0]
    p = t.reshape(r, D // 32, 2, L).transpose(0, 1, 3, 2)
    p = p.astype(jnp.bfloat16).reshape(r * (D // 2), 2)
    return lax.bitcast_convert_type(p, jnp.int32)


@jax.jit
def kernel(text_feats, weight, comp_residual, attr_residual, obj_residual,
           attr_idx, obj_idx):
    run = pl.kernel(
        _body,
        mesh=plsc.VectorSubcoreMesh(core_axis_name="c", subcore_axis_name="s"),
        compiler_params=pltpu.CompilerParams(needs_layout_passes=False),
        out_type=jax.ShapeDtypeStruct((NUM_COMPS * D,), jnp.float32),
        scratch_types=[
            pltpu.VMEM((NUM_ATTRS * D // 2,), jnp.int32),
            pltpu.VMEM((NUM_OBJS * D // 2,), jnp.int32),
        ] + 2 * [
            pltpu.VMEM((CHUNK * D,), jnp.float32),
            pltpu.VMEM((CHUNK * D,), jnp.float32),
            pltpu.VMEM((CHUNK * D,), jnp.float32),
            pltpu.VMEM((CHUNK,), jnp.float32),
            pltpu.VMEM((CHUNK,), jnp.int32),
            pltpu.VMEM((CHUNK,), jnp.int32),
        ] + 4 * [pltpu.SemaphoreType.DMA],
    )
    out = run(text_feats.reshape(-1), weight, comp_residual.reshape(-1),
              _pack_table(attr_residual), _pack_table(obj_residual),
              attr_idx, obj_idx)
    return out.reshape(NUM_COMPS, D)


# speedup vs baseline: 1.1862x; 1.1862x over previous
import jax
import jax.numpy as jnp
from jax import lax
from jax.experimental import pallas as pl
from jax.experimental.pallas import tpu as pltpu
from jax.experimental.pallas import tpu_sc as plsc

NUM_COMPS = 100000
NUM_ATTRS = 200
NUM_OBJS = 500
D = 128
L = 16
NC = 2
NS = 16
NW = NC * NS
CHUNK = 80
NCHUNKS = NUM_COMPS // CHUNK
NSLOTS = (NCHUNKS + NW - 1) // NW
RPB = 4


def _rsqrt16(x):
    xi = plsc.bitcast(x, jnp.int32)
    yi = jnp.int32(0x5F3759DF) - lax.shift_right_logical(xi, 1)
    y = plsc.bitcast(yi, jnp.float32)
    for _ in range(2):
        y = y * (1.5 - 0.5 * x * y * y)
    return y


def _body(text_hbm, w_hbm, comp_hbm, attr_hbm, obj_hbm, ai_hbm, oi_hbm,
          out_hbm, attr_v, obj_v,
          tx0, cp0, ou0, wv0, av0, ov0,
          tx1, cp1, ou1, wv1, av1, ov1,
          isem0, isem1, osem0, osem1):
    wid = lax.axis_index("s") * NC + lax.axis_index("c")

    pltpu.sync_copy(attr_hbm, attr_v)
    pltpu.sync_copy(obj_hbm, obj_v)

    bufs = ((tx0, cp0, ou0, wv0, av0, ov0, isem0, osem0),
            (tx1, cp1, ou1, wv1, av1, ov1, isem1, osem1))

    def start_in(s, b):
        tx, cp, _, wv, av, ov, isem, _ = bufs[b]
        cid = wid + NW * s
        base = cid * (CHUNK * D)
        sbase = cid * CHUNK
        pltpu.async_copy(text_hbm.at[pl.ds(base, CHUNK * D)], tx, isem)
        pltpu.async_copy(comp_hbm.at[pl.ds(base, CHUNK * D)], cp, isem)
        pltpu.async_copy(w_hbm.at[pl.ds(sbase, CHUNK)], wv, isem)
        pltpu.async_copy(ai_hbm.at[pl.ds(sbase, CHUNK)], av, isem)
        pltpu.async_copy(oi_hbm.at[pl.ds(sbase, CHUNK)], ov, isem)

    def wait_in(b):
        tx, cp, _, wv, av, ov, isem, _ = bufs[b]
        pltpu.make_async_copy(text_hbm.at[pl.ds(0, CHUNK * D)], tx, isem).wait()
        pltpu.make_async_copy(comp_hbm.at[pl.ds(0, CHUNK * D)], cp, isem).wait()
        pltpu.make_async_copy(w_hbm.at[pl.ds(0, CHUNK)], wv, isem).wait()
        pltpu.make_async_copy(ai_hbm.at[pl.ds(0, CHUNK)], av, isem).wait()
        pltpu.make_async_copy(oi_hbm.at[pl.ds(0, CHUNK)], ov, isem).wait()

    def start_out(s, b):
        ou, osem = bufs[b][2], bufs[b][7]
        base = (wid + NW * s) * (CHUNK * D)
        pltpu.async_copy(ou, out_hbm.at[pl.ds(base, CHUNK * D)], osem)

    def wait_out(b):
        ou, osem = bufs[b][2], bufs[b][7]
        pltpu.make_async_copy(ou, out_hbm.at[pl.ds(0, CHUNK * D)], osem).wait()

    def compute(b):
        tx, cp, ou, wv, av, ov, _, _ = bufs[b]

        def do_group(g, carry):
            w16 = wv[pl.ds(g * L, L)]
            ai16 = av[pl.ds(g * L, L)] * (D // 2)
            oi16 = ov[pl.ds(g * L, L)] * (D // 2)
            ws = [w16[j] for j in range(L)]
            obases = [oi16[j] for j in range(L)]

            def unpack32(ref, off):
                return plsc.unpack(
                    plsc.bitcast(ref[pl.ds(off, L)], jnp.bfloat16),
                    format=plsc.PackFormat.INTERLEAVED,
                    preferred_element_type=jnp.float32)

            def block(j, shared_ar, abases):
                rows = tuple(range(j, j + RPB))
                hws = [0.5 * ws[r] for r in rows]
                rbs = [(g * L + r) * D for r in rows]
                us = [[] for _ in rows]
                accs = [jnp.zeros((L,), jnp.float32) for _ in rows]
                for m in range(D // 32):
                    obs = [unpack32(obj_v, obases[r] + m * L) for r in rows]
                    if shared_ar is None:
                        ars = [unpack32(attr_v, abases[r] + m * L)
                               for r in rows]
                    else:
                        ars = [shared_ar[m]] * RPB
                    for h in range(2):
                        for i, r in enumerate(rows):
                            off = rbs[i] + m * 32 + h * L
                            ta = tx[pl.ds(off, L)]
                            co = cp[pl.ds(off, L)]
                            u = ta + ws[r] * co + hws[i] * (ars[i][h] + obs[i][h])
                            accs[i] = accs[i] + u * u
                            us[i].append(u)
                ssqs = [jnp.maximum(jnp.sum(a), 1e-24) for a in accs]
                rvs = [_rsqrt16(jnp.full((L,), s, jnp.float32)) for s in ssqs]
                for k in range(D // L):
                    for i in range(len(rows)):
                        ou[pl.ds(rbs[i] + k * L, L)] = us[i][k] * rvs[i]

            abases = [ai16[j] for j in range(L)]
            for j in range(0, L, RPB):
                block(j, None, abases)

            return carry

        lax.fori_loop(0, CHUNK // L, do_group, 0)

    def valid(s):
        return wid + NW * s < NCHUNKS

    start_in(0, 0)

    def pair(p, carry):
        s0 = 2 * p

        @pl.when(valid(s0 + 1))
        def _():
            start_in(s0 + 1, 1)

        @pl.when(valid(s0))
        def _():
            wait_in(0)

            @pl.when(p > 0)
            def _():
                wait_out(0)

            compute(0)
            start_out(s0, 0)

        @pl.when(valid(s0 + 2))
        def _():
            start_in(s0 + 2, 0)

        @pl.when(valid(s0 + 1))
        def _():
            wait_in(1)

            @pl.when(p > 0)
            def _():
                wait_out(1)

            compute(1)
            start_out(s0 + 1, 1)

        return carry

    lax.fori_loop(0, NSLOTS // 2, pair, 0)
    wait_out(0)
    wait_out(1)


def _pack_table(t):
    r = t.shape[0]
    p = t.reshape(r, D // 32, 2, L).transpose(0, 1, 3, 2)
    p = p.astype(jnp.bfloat16).reshape(r * (D // 2), 2)
    return lax.bitcast_convert_type(p, jnp.int32)


@jax.jit
def kernel(text_feats, weight, comp_residual, attr_residual, obj_residual,
           attr_idx, obj_idx):
    run = pl.kernel(
        _body,
        mesh=plsc.VectorSubcoreMesh(core_axis_name="c", subcore_axis_name="s"),
        compiler_params=pltpu.CompilerParams(needs_layout_passes=False),
        out_type=jax.ShapeDtypeStruct((NUM_COMPS * D,), jnp.float32),
        scratch_types=[
            pltpu.VMEM((NUM_ATTRS * D // 2,), jnp.int32),
            pltpu.VMEM((NUM_OBJS * D // 2,), jnp.int32),
        ] + 2 * [
            pltpu.VMEM((CHUNK * D,), jnp.float32),
            pltpu.VMEM((CHUNK * D,), jnp.float32),
            pltpu.VMEM((CHUNK * D,), jnp.float32),
            pltpu.VMEM((CHUNK,), jnp.float32),
            pltpu.VMEM((CHUNK,), jnp.int32),
            pltpu.VMEM((CHUNK,), jnp.int32),
        ] + 4 * [pltpu.SemaphoreType.DMA],
    )
    out = run(text_feats.reshape(-1), weight, comp_residual.reshape(-1),
              _pack_table(attr_residual), _pack_table(obj_residual),
              attr_idx, obj_idx)
    return out.reshape(NUM_COMPS, D)


# speedup vs baseline: 1.2824x; 1.0811x over previous
import jax
import jax.numpy as jnp
from jax import lax
from jax.experimental import pallas as pl
from jax.experimental.pallas import tpu as pltpu
from jax.experimental.pallas import tpu_sc as plsc

NUM_COMPS = 100000
NUM_ATTRS = 200
NUM_OBJS = 500
D = 128
L = 16
NC = 2
NS = 16
NW = NC * NS
CHUNK = 80
NCHUNKS = NUM_COMPS // CHUNK
NSLOTS = (NCHUNKS + NW - 1) // NW
RPB = 8


def _rsqrt16(x):
    xi = plsc.bitcast(x, jnp.int32)
    yi = jnp.int32(0x5F3759DF) - lax.shift_right_logical(xi, 1)
    y = plsc.bitcast(yi, jnp.float32)
    for _ in range(2):
        y = y * (1.5 - 0.5 * x * y * y)
    return y


def _body(text_hbm, w_hbm, comp_hbm, attr_hbm, obj_hbm, ai_hbm, oi_hbm,
          out_hbm, attr_v, obj_v,
          tx0, cp0, ou0, wv0, av0, ov0,
          tx1, cp1, ou1, wv1, av1, ov1,
          isem0, isem1, osem0, osem1):
    wid = lax.axis_index("s") * NC + lax.axis_index("c")

    pltpu.sync_copy(attr_hbm, attr_v)
    pltpu.sync_copy(obj_hbm, obj_v)

    bufs = ((tx0, cp0, ou0, wv0, av0, ov0, isem0, osem0),
            (tx1, cp1, ou1, wv1, av1, ov1, isem1, osem1))

    def start_in(s, b):
        tx, cp, _, wv, av, ov, isem, _ = bufs[b]
        cid = wid + NW * s
        base = cid * (CHUNK * D)
        sbase = cid * CHUNK
        pltpu.async_copy(text_hbm.at[pl.ds(base, CHUNK * D)], tx, isem)
        pltpu.async_copy(comp_hbm.at[pl.ds(base, CHUNK * D)], cp, isem)
        pltpu.async_copy(w_hbm.at[pl.ds(sbase, CHUNK)], wv, isem)
        pltpu.async_copy(ai_hbm.at[pl.ds(sbase, CHUNK)], av, isem)
        pltpu.async_copy(oi_hbm.at[pl.ds(sbase, CHUNK)], ov, isem)

    def wait_in(b):
        tx, cp, _, wv, av, ov, isem, _ = bufs[b]
        pltpu.make_async_copy(text_hbm.at[pl.ds(0, CHUNK * D)], tx, isem).wait()
        pltpu.make_async_copy(comp_hbm.at[pl.ds(0, CHUNK * D)], cp, isem).wait()
        pltpu.make_async_copy(w_hbm.at[pl.ds(0, CHUNK)], wv, isem).wait()
        pltpu.make_async_copy(ai_hbm.at[pl.ds(0, CHUNK)], av, isem).wait()
        pltpu.make_async_copy(oi_hbm.at[pl.ds(0, CHUNK)], ov, isem).wait()

    def start_out(s, b):
        ou, osem = bufs[b][2], bufs[b][7]
        base = (wid + NW * s) * (CHUNK * D)
        pltpu.async_copy(ou, out_hbm.at[pl.ds(base, CHUNK * D)], osem)

    def wait_out(b):
        ou, osem = bufs[b][2], bufs[b][7]
        pltpu.make_async_copy(ou, out_hbm.at[pl.ds(0, CHUNK * D)], osem).wait()

    def compute(b):
        tx, cp, ou, wv, av, ov, _, _ = bufs[b]

        def do_group(g, carry):
            w16 = wv[pl.ds(g * L, L)]
            ai16 = av[pl.ds(g * L, L)] * (D // 2)
            oi16 = ov[pl.ds(g * L, L)] * (D // 2)
            ws = [w16[j] for j in range(L)]
            obases = [oi16[j] for j in range(L)]

            def unpack32(ref, off):
                return plsc.unpack(
                    plsc.bitcast(ref[pl.ds(off, L)], jnp.bfloat16),
                    format=plsc.PackFormat.INTERLEAVED,
                    preferred_element_type=jnp.float32)

            def block(j, shared_ar, abases):
                rows = tuple(range(j, j + RPB))
                hws = [0.5 * ws[r] for r in rows]
                rbs = [(g * L + r) * D for r in rows]
                us = [[] for _ in rows]
                accs = [jnp.zeros((L,), jnp.float32) for _ in rows]
                for m in range(D // 32):
                    obs = [unpack32(obj_v, obases[r] + m * L) for r in rows]
                    if shared_ar is None:
                        ars = [unpack32(attr_v, abases[r] + m * L)
                               for r in rows]
                    else:
                        ars = [shared_ar[m]] * RPB
                    for h in range(2):
                        for i, r in enumerate(rows):
                            off = rbs[i] + m * 32 + h * L
                            ta = tx[pl.ds(off, L)]
                            co = cp[pl.ds(off, L)]
                            u = ta + ws[r] * co + hws[i] * (ars[i][h] + obs[i][h])
                            accs[i] = accs[i] + u * u
                            us[i].append(u)
                ssqs = [jnp.maximum(jnp.sum(a), 1e-24) for a in accs]
                rvs = [_rsqrt16(jnp.full((L,), s, jnp.float32)) for s in ssqs]
                for k in range(D // L):
                    for i in range(len(rows)):
                        ou[pl.ds(rbs[i] + k * L, L)] = us[i][k] * rvs[i]

            abases = [ai16[j] for j in range(L)]
            for j in range(0, L, RPB):
                block(j, None, abases)

            return carry

        lax.fori_loop(0, CHUNK // L, do_group, 0)

    def valid(s):
        return wid + NW * s < NCHUNKS

    start_in(0, 0)

    def pair(p, carry):
        s0 = 2 * p

        @pl.when(valid(s0 + 1))
        def _():
            start_in(s0 + 1, 1)

        @pl.when(valid(s0))
        def _():
            wait_in(0)

            @pl.when(p > 0)
            def _():
                wait_out(0)

            compute(0)
            start_out(s0, 0)

        @pl.when(valid(s0 + 2))
        def _():
            start_in(s0 + 2, 0)

        @pl.when(valid(s0 + 1))
        def _():
            wait_in(1)

            @pl.when(p > 0)
            def _():
                wait_out(1)

            compute(1)
            start_out(s0 + 1, 1)

        return carry

    lax.fori_loop(0, NSLOTS // 2, pair, 0)
    wait_out(0)
    wait_out(1)


def _pack_table(t):
    r = t.shape[0]
    p = t.reshape(r, D // 32, 2, L).transpose(0, 1, 3, 2)
    p = p.astype(jnp.bfloat16).reshape(r * (D // 2), 2)
    return lax.bitcast_convert_type(p, jnp.int32)


@jax.jit
def kernel(text_feats, weight, comp_residual, attr_residual, obj_residual,
           attr_idx, obj_idx):
    run = pl.kernel(
        _body,
        mesh=plsc.VectorSubcoreMesh(core_axis_name="c", subcore_axis_name="s"),
        compiler_params=pltpu.CompilerParams(needs_layout_passes=False),
        out_type=jax.ShapeDtypeStruct((NUM_COMPS * D,), jnp.float32),
        scratch_types=[
            pltpu.VMEM((NUM_ATTRS * D // 2,), jnp.int32),
            pltpu.VMEM((NUM_OBJS * D // 2,), jnp.int32),
        ] + 2 * [
            pltpu.VMEM((CHUNK * D,), jnp.float32),
            pltpu.VMEM((CHUNK * D,), jnp.float32),
            pltpu.VMEM((CHUNK * D,), jnp.float32),
            pltpu.VMEM((CHUNK,), jnp.float32),
            pltpu.VMEM((CHUNK,), jnp.int32),
            pltpu.VMEM((CHUNK,), jnp.int32),
        ] + 4 * [pltpu.SemaphoreType.DMA],
    )
    out = run(text_feats.reshape(-1), weight, comp_residual.reshape(-1),
              _pack_table(attr_residual), _pack_table(obj_residual),
              attr_idx, obj_idx)
    return out.reshape(NUM_COMPS, D)
